# Initial kernel scaffold; baseline (speedup 1.0000x reference)
import jax
import jax.numpy as jnp
from jax.experimental import pallas as pl


def kernel(embed, indices):
    raise NotImplementedError("write your pallas kernel here")


# speedup vs baseline: 1.1076x; 1.1076x over previous
import functools

import jax
import jax.numpy as jnp
from jax import lax
from jax.experimental import pallas as pl
from jax.experimental.pallas import tpu as pltpu
from jax.experimental.pallas import tpu_sc as plsc

_NC = 2
_NS = 16
_NW = _NC * _NS

_ROWS = 1_000_000
_D = 32
_B = 16384 * 100
_B_PER_W = _B // _NW
_CHUNK = 2048
_NCHUNKS = _B_PER_W // _CHUNK


def _gather_kernel(table_hbm, idx_hbm, out_hbm, idx_v, rows_v, sem):
    wid = lax.axis_index("s") * _NC + lax.axis_index("c")
    base0 = wid * _B_PER_W

    @pl.loop(0, _NCHUNKS)
    def _chunk(i):
        base = base0 + i * _CHUNK
        pltpu.sync_copy(idx_hbm.at[pl.ds(base, _CHUNK)], idx_v)
        pltpu.async_copy(table_hbm.at[idx_v], rows_v, sem).wait()
        pltpu.sync_copy(rows_v, out_hbm.at[pl.ds(base, _CHUNK)])


@jax.jit
def _lookup(embed, idx_flat):
    mesh = plsc.VectorSubcoreMesh(
        core_axis_name="c", subcore_axis_name="s",
        num_cores=_NC, num_subcores=_NS)
    return pl.kernel(
        _gather_kernel,
        out_type=jax.ShapeDtypeStruct((_B, _D), jnp.float32),
        mesh=mesh,
        scratch_types=[
            pltpu.VMEM((_CHUNK,), jnp.int32),
            pltpu.VMEM((_CHUNK, _D), jnp.float32),
            pltpu.SemaphoreType.DMA,
        ],
        compiler_params=pltpu.CompilerParams(use_tc_tiling_on_sc=False),
    )(embed, idx_flat)


def kernel(embed, indices):
    idx_flat = indices.reshape(-1).astype(jnp.int32)
    out = _lookup(embed, idx_flat)
    return out.reshape(indices.shape + (_D,))


# speedup vs baseline: 1.1109x; 1.0030x over previous
import jax
import jax.numpy as jnp
from jax import lax
from jax.experimental import pallas as pl
from jax.experimental.pallas import tpu as pltpu
from jax.experimental.pallas import tpu_sc as plsc

_NC = 2
_NS = 16
_NW = _NC * _NS

_D = 32
_B = 16384 * 100
_B_PER_W = _B // _NW
_CHUNK = 1600
_NBUF = 2
_NCHUNKS = _B_PER_W // _CHUNK


def _gather_kernel(table_hbm, idx_hbm, out_hbm,
                   idx0, idx1, rows0, rows1,
                   isem0, isem1, gsem0, gsem1, wsem0, wsem1):
    idxs = (idx0, idx1)
    rows = (rows0, rows1)
    isems = (isem0, isem1)
    gsems = (gsem0, gsem1)
    wsems = (wsem0, wsem1)

    wid = lax.axis_index("s") * _NC + lax.axis_index("c")
    base0 = wid * _B_PER_W

    for b in range(_NBUF):
        pltpu.async_copy(
            idx_hbm.at[pl.ds(base0 + b * _CHUNK, _CHUNK)], idxs[b], isems[b])

    @pl.loop(0, _NCHUNKS, step=_NBUF)
    def _group(i0):
        for b in range(_NBUF):
            i = i0 + b
            base = base0 + i * _CHUNK

            @pl.when(i0 >= _NBUF)
            def _():
                pltpu.make_async_copy(
                    rows[b],
                    out_hbm.at[pl.ds(base - _NBUF * _CHUNK, _CHUNK)],
                    wsems[b]).wait()

            pltpu.make_async_copy(
                idx_hbm.at[pl.ds(base, _CHUNK)], idxs[b], isems[b]).wait()
            pltpu.async_copy(table_hbm.at[idxs[b]], rows[b], gsems[b]).wait()

            pltpu.async_copy(
                rows[b], out_hbm.at[pl.ds(base, _CHUNK)], wsems[b])

            @pl.when(i0 + 2 * _NBUF <= _NCHUNKS)
            def _():
                pltpu.async_copy(
                    idx_hbm.at[pl.ds(base + _NBUF * _CHUNK, _CHUNK)],
                    idxs[b], isems[b])

    for b in range(_NBUF):
        i = _NCHUNKS - _NBUF + b
        pltpu.make_async_copy(
            rows[b], out_hbm.at[pl.ds(base0 + i * _CHUNK, _CHUNK)],
            wsems[b]).wait()


@jax.jit
def _lookup(embed, idx_flat):
    mesh = plsc.VectorSubcoreMesh(
        core_axis_name="c", subcore_axis_name="s",
        num_cores=_NC, num_subcores=_NS)
    return pl.kernel(
        _gather_kernel,
        out_type=jax.ShapeDtypeStruct((_B, _D), jnp.float32),
        mesh=mesh,
        scratch_types=[
            pltpu.VMEM((_CHUNK,), jnp.int32),
            pltpu.VMEM((_CHUNK,), jnp.int32),
            pltpu.VMEM((_CHUNK, _D), jnp.float32),
            pltpu.VMEM((_CHUNK, _D), jnp.float32),
            pltpu.SemaphoreType.DMA,
            pltpu.SemaphoreType.DMA,
            pltpu.SemaphoreType.DMA,
            pltpu.SemaphoreType.DMA,
            pltpu.SemaphoreType.DMA,
            pltpu.SemaphoreType.DMA,
        ],
        compiler_params=pltpu.CompilerParams(use_tc_tiling_on_sc=False),
    )(embed, idx_flat)


def kernel(embed, indices):
    idx_flat = indices.reshape(-1).astype(jnp.int32)
    out = _lookup(embed, idx_flat)
    return out.reshape(indices.shape + (_D,))


# speedup vs baseline: 1.1114x; 1.0005x over previous
import jax
import jax.numpy as jnp
from jax import lax
from jax.experimental import pallas as pl
from jax.experimental.pallas import tpu as pltpu
from jax.experimental.pallas import tpu_sc as plsc

_NC = 2
_NS = 16
_NW = _NC * _NS

_D = 32
_B = 16384 * 100
_B_PER_W = _B // _NW
_K = 4
_CHUNK = 400
_GROUP = _K * _CHUNK
_NGROUPS = _B_PER_W // _GROUP


def _gather_kernel(table_hbm, idx_hbm, out_hbm, *scratch):
    idxs = [scratch[0:_K], scratch[_K:2 * _K]]
    rows = [scratch[2 * _K:3 * _K], scratch[3 * _K:4 * _K]]
    isem = scratch[4 * _K:4 * _K + 2]
    gsem = scratch[4 * _K + 2:4 * _K + 4]
    wsem = scratch[4 * _K + 4:4 * _K + 6]

    wid = lax.axis_index("s") * _NC + lax.axis_index("c")
    base0 = wid * _B_PER_W

    def cbase(g, j):
        return base0 + g * _GROUP + j * _CHUNK

    def idx_copy(g, j, s):
        return pltpu.make_async_copy(
            idx_hbm.at[pl.ds(cbase(g, j), _CHUNK)], idxs[s][j], isem[s])

    def out_copy(g, j, s):
        return pltpu.make_async_copy(
            rows[s][j], out_hbm.at[pl.ds(cbase(g, j), _CHUNK)], wsem[s])

    for g in (0, 1):
        for j in range(_K):
            idx_copy(g, j, g).start()

    @pl.loop(0, _NGROUPS, step=2)
    def _pair(g0):
        for s in range(2):
            g = g0 + s

            @pl.when(g0 >= 2)
            def _():
                for j in range(_K):
                    out_copy(g - 2, j, s).wait()

            for j in range(_K):
                idx_copy(g, j, s).wait()

            gathers = [
                pltpu.async_copy(table_hbm.at[idxs[s][j]], rows[s][j], gsem[s])
                for j in range(_K)]

            for j in range(_K):
                gathers[j].wait()
                out_copy(g, j, s).start()

            @pl.when(g0 + 4 <= _NGROUPS)
            def _():
                for j in range(_K):
                    idx_copy(g + 2, j, s).start()

    for g in (_NGROUPS - 2, _NGROUPS - 1):
        for j in range(_K):
            out_copy(g, j, g % 2).wait()


@jax.jit
def _lookup(embed, idx_flat):
    mesh = plsc.VectorSubcoreMesh(
        core_axis_name="c", subcore_axis_name="s",
        num_cores=_NC, num_subcores=_NS)
    return pl.kernel(
        _gather_kernel,
        out_type=jax.ShapeDtypeStruct((_B, _D), jnp.float32),
        mesh=mesh,
        scratch_types=(
            [pltpu.VMEM((_CHUNK,), jnp.int32) for _ in range(2 * _K)]
            + [pltpu.VMEM((_CHUNK, _D), jnp.float32) for _ in range(2 * _K)]
            + [pltpu.SemaphoreType.DMA for _ in range(6)]
        ),
        compiler_params=pltpu.CompilerParams(use_tc_tiling_on_sc=False),
    )(embed, idx_flat)


def kernel(embed, indices):
    idx_flat = indices.reshape(-1).astype(jnp.int32)
    out = _lookup(embed, idx_flat)
    return out.reshape(indices.shape + (_D,))


# speedup vs baseline: 3.2306x; 2.9067x over previous
import jax
import jax.numpy as jnp
from jax import lax
from jax.experimental import pallas as pl
from jax.experimental.pallas import tpu as pltpu
from jax.experimental.pallas import tpu_sc as plsc

_NC = 2
_NS = 16
_NW = _NC * _NS

_D = 32
_BATCH = 16384
_NPOS = 100
_W = _BATCH // _NW
_L = 16


def _gather_kernel(table_hbm, idx_hbm, out_hbm, *scratch):
    idxs = scratch[0:2]
    rows = scratch[2:4]
    valst = scratch[4:6]
    isem = scratch[6:8]
    gsem = scratch[8:10]
    wsem = scratch[10:12]

    wid = lax.axis_index("s") * _NC + lax.axis_index("c")
    b0 = wid * _W

    def idx_copy(p, s):
        return pltpu.make_async_copy(
            idx_hbm.at[p, pl.ds(b0, _W)], idxs[s], isem[s])

    def out_copy(p, s):
        return pltpu.make_async_copy(
            valst[s], out_hbm.at[p, :, pl.ds(b0, _W)], wsem[s])

    iota = lax.iota(jnp.int32, _L)

    for s in range(2):
        idx_copy(s, s).start()

    @pl.loop(0, _NPOS, step=2)
    def _pair(p0):
        for s in range(2):
            p = p0 + s

            @pl.when(p0 >= 2)
            def _():
                out_copy(p - 2, s).wait()

            idx_copy(p, s).wait()
            pltpu.async_copy(table_hbm.at[idxs[s]], rows[s], gsem[s]).wait()

            @pl.when(p0 + 4 <= _NPOS)
            def _():
                idx_copy(p + 2, s).start()

            if True:
                @pl.loop(0, _W, step=_L)
                def _blk(bb):
                    ridx = bb + iota
                    for c in range(_D):
                        v = plsc.load_gather(
                            rows[s], [ridx, jnp.full((_L,), c, jnp.int32)])
                        valst[s][c, pl.ds(bb, _L)] = v

            out_copy(p, s).start()

    for s in range(2):
        out_copy(_NPOS - 2 + s, s).wait()


@jax.jit
def _lookup(embed, idx_t):
    mesh = plsc.VectorSubcoreMesh(
        core_axis_name="c", subcore_axis_name="s",
        num_cores=_NC, num_subcores=_NS)
    return pl.kernel(
        _gather_kernel,
        out_type=jax.ShapeDtypeStruct((_NPOS, _D, _BATCH), jnp.float32),
        mesh=mesh,
        scratch_types=(
            [pltpu.VMEM((_W,), jnp.int32) for _ in range(2)]
            + [pltpu.VMEM((_W, _D), jnp.float32) for _ in range(2)]
            + [pltpu.VMEM((_D, _W), jnp.float32) for _ in range(2)]
            + [pltpu.SemaphoreType.DMA for _ in range(6)]
        ),
        compiler_params=pltpu.CompilerParams(use_tc_tiling_on_sc=False, needs_layout_passes=False),
    )(embed, idx_t)


def kernel(embed, indices):
    idx_t = jnp.swapaxes(indices, 0, 1).astype(jnp.int32)
    out = _lookup(embed, idx_t)
    return out.transpose(2, 0, 1)


# speedup vs baseline: 3.4852x; 1.0788x over previous
import jax
import jax.numpy as jnp
from jax import lax
from jax.experimental import pallas as pl
from jax.experimental.pallas import tpu as pltpu
from jax.experimental.pallas import tpu_sc as plsc

_NC = 2
_NS = 16
_NW = _NC * _NS

_D = 32
_BATCH = 16384
_NPOS = 100
_W = _BATCH // _NW
_L = 16


def _gather_kernel(table_hbm, idx_hbm, out_hbm, *scratch):
    idxs = scratch[0:2]
    rows = scratch[2:4]
    valst = scratch[4:6]
    isem = scratch[6:8]
    gsem = scratch[8:10]
    wsem = scratch[10:12]

    wid = lax.axis_index("s") * _NC + lax.axis_index("c")
    b0 = wid * _W

    def idx_copy(p, s):
        return pltpu.make_async_copy(
            idx_hbm.at[p, pl.ds(b0, _W)], idxs[s], isem[s])

    def gather(s):
        return pltpu.make_async_copy(table_hbm.at[idxs[s]], rows[s], gsem[s])

    def out_copy(p, s):
        return pltpu.make_async_copy(
            valst[s], out_hbm.at[p, :, pl.ds(b0, _W)], wsem[s])

    iota = lax.iota(jnp.int32, _L)

    for s in range(2):
        idx_copy(s, s).start()
    idx_copy(0, 0).wait()
    gather(0).start()

    @pl.loop(0, _NPOS, step=2)
    def _pair(p0):
        for s in range(2):
            p = p0 + s

            gather(s).wait()

            @pl.when(p0 + 4 <= _NPOS)
            def _():
                idx_copy(p + 2, s).start()

            @pl.when(p + 1 < _NPOS)
            def _():
                idx_copy(p + 1, 1 - s).wait()
                gather(1 - s).start()

            @pl.when(p0 >= 2)
            def _():
                out_copy(p - 2, s).wait()

            @pl.loop(0, _W, step=_L)
            def _blk(bb):
                ridx = bb + iota
                for c in range(_D):
                    v = plsc.load_gather(
                        rows[s], [ridx, jnp.full((_L,), c, jnp.int32)])
                    valst[s][c, pl.ds(bb, _L)] = v

            out_copy(p, s).start()

    for s in range(2):
        out_copy(_NPOS - 2 + s, s).wait()


@jax.jit
def _lookup(embed, idx_t):
    mesh = plsc.VectorSubcoreMesh(
        core_axis_name="c", subcore_axis_name="s",
        num_cores=_NC, num_subcores=_NS)
    return pl.kernel(
        _gather_kernel,
        out_type=jax.ShapeDtypeStruct((_NPOS, _D, _BATCH), jnp.float32),
        mesh=mesh,
        scratch_types=(
            [pltpu.VMEM((_W,), jnp.int32) for _ in range(2)]
            + [pltpu.VMEM((_W, _D), jnp.float32) for _ in range(2)]
            + [pltpu.VMEM((_D, _W), jnp.float32) for _ in range(2)]
            + [pltpu.SemaphoreType.DMA for _ in range(6)]
        ),
        compiler_params=pltpu.CompilerParams(
            use_tc_tiling_on_sc=False, needs_layout_passes=False),
    )(embed, idx_t)


def kernel(embed, indices):
    idx_t = jnp.swapaxes(indices, 0, 1).astype(jnp.int32)
    out = _lookup(embed, idx_t)
    return out.transpose(2, 0, 1)


# speedup vs baseline: 4.7419x; 1.3606x over previous
import jax
import jax.numpy as jnp
from jax import lax
from jax.experimental import pallas as pl
from jax.experimental.pallas import tpu as pltpu
from jax.experimental.pallas import tpu_sc as plsc

_NC = 2
_NS = 16
_NW = _NC * _NS

_D = 32
_BATCH = 16384
_NPOS = 100
_W = _BATCH // _NW
_L = 16


def _gather_kernel(table_hbm, idx_hbm, out_hbm, *scratch):
    idxs = scratch[0:2]
    rows = scratch[2:4]
    valst = scratch[4:6]
    isem = scratch[6:8]
    gsem = scratch[8:10]
    wsem = scratch[10:12]

    wid = lax.axis_index("s") * _NC + lax.axis_index("c")
    b0 = wid * _W

    def idx_copy(p, s):
        return pltpu.make_async_copy(
            idx_hbm.at[p, pl.ds(b0, _W)], idxs[s], isem[s])

    def gather(s):
        return pltpu.make_async_copy(table_hbm.at[idxs[s]], rows[s], gsem[s])

    def out_copy(p, s):
        return pltpu.make_async_copy(
            valst[s], out_hbm.at[p, :, pl.ds(b0, _W)], wsem[s])

    iota = lax.iota(jnp.int32, _L)

    for s in range(2):
        idx_copy(s, s).start()
    idx_copy(0, 0).wait()
    gather(0).start()

    @pl.loop(0, _NPOS, step=2)
    def _pair(p0):
        for s in range(2):
            p = p0 + s

            gather(s).wait()

            @pl.when(p0 + 4 <= _NPOS)
            def _():
                idx_copy(p + 2, s).start()

            @pl.when(p + 1 < _NPOS)
            def _():
                idx_copy(p + 1, 1 - s).wait()
                gather(1 - s).start()

            @pl.when(p0 >= 2)
            def _():
                out_copy(p - 2, s).wait()

            @plsc.parallel_loop(0, _W, _L, unroll=2)
            def _blk(bb):
                ridx = bb + iota
                for c in range(_D):
                    v = plsc.load_gather(
                        rows[s], [ridx, jnp.full((_L,), c, jnp.int32)])
                    valst[s][c, pl.ds(bb, _L)] = v

            out_copy(p, s).start()

    for s in range(2):
        out_copy(_NPOS - 2 + s, s).wait()


@jax.jit
def _lookup(embed, idx_t):
    mesh = plsc.VectorSubcoreMesh(
        core_axis_name="c", subcore_axis_name="s",
        num_cores=_NC, num_subcores=_NS)
    return pl.kernel(
        _gather_kernel,
        out_type=jax.ShapeDtypeStruct((_NPOS, _D, _BATCH), jnp.float32),
        mesh=mesh,
        scratch_types=(
            [pltpu.VMEM((_W,), jnp.int32) for _ in range(2)]
            + [pltpu.VMEM((_W, _D), jnp.float32) for _ in range(2)]
            + [pltpu.VMEM((_D, _W), jnp.float32) for _ in range(2)]
            + [pltpu.SemaphoreType.DMA for _ in range(6)]
        ),
        compiler_params=pltpu.CompilerParams(
            use_tc_tiling_on_sc=False, needs_layout_passes=False),
    )(embed, idx_t)


def kernel(embed, indices):
    idx_t = jnp.swapaxes(indices, 0, 1).astype(jnp.int32)
    out = _lookup(embed, idx_t)
    return out.transpose(2, 0, 1)


# speedup vs baseline: 5.5988x; 1.1807x over previous
import jax
import jax.numpy as jnp
from jax import lax
from jax.experimental import pallas as pl
from jax.experimental.pallas import tpu as pltpu
from jax.experimental.pallas import tpu_sc as plsc

_NC = 2
_NS = 16
_NW = _NC * _NS

_D = 32
_BATCH = 16384
_NPOS = 100
_W = _BATCH // _NW
_L = 16


def _gather_kernel(table_hbm, idx_hbm, out_hbm, *scratch):
    idxs = scratch[0:2]
    rows = scratch[2:4]
    valst = scratch[4:6]
    isem = scratch[6:8]
    gsem = scratch[8:10]
    wsem = scratch[10:12]

    wid = lax.axis_index("s") * _NC + lax.axis_index("c")
    b0 = wid * _W

    def idx_copy(p, s):
        return pltpu.make_async_copy(
            idx_hbm.at[p, pl.ds(b0, _W)], idxs[s], isem[s])

    def gather(s):
        return pltpu.make_async_copy(table_hbm.at[idxs[s]], rows[s], gsem[s])

    tcb = wid * (_W // 128)

    def out_copy(p, s):
        return pltpu.make_async_copy(
            valst[s], out_hbm.at[p, :, pl.ds(tcb, _W // 128), :, :], wsem[s])

    iota = lax.iota(jnp.int32, _L)

    for s in range(2):
        idx_copy(s, s).start()
    idx_copy(0, 0).wait()
    gather(0).start()

    @pl.loop(0, _NPOS, step=2)
    def _pair(p0):
        for s in range(2):
            p = p0 + s

            gather(s).wait()

            @pl.when(p0 + 4 <= _NPOS)
            def _():
                idx_copy(p + 2, s).start()

            @pl.when(p + 1 < _NPOS)
            def _():
                idx_copy(p + 1, 1 - s).wait()
                gather(1 - s).start()

            @pl.when(p0 >= 2)
            def _():
                out_copy(p - 2, s).wait()

            @plsc.parallel_loop(0, _W, _L, unroll=2)
            def _blk(bb):
                ridx = bb + iota
                tcl = bb // 128
                cc0 = bb % 128
                for c in range(_D):
                    v = plsc.load_gather(
                        rows[s], [ridx, jnp.full((_L,), c, jnp.int32)])
                    valst[s][c // 8, tcl, c % 8, pl.ds(cc0, _L)] = v

            out_copy(p, s).start()

    for s in range(2):
        out_copy(_NPOS - 2 + s, s).wait()


@jax.jit
def _lookup(embed, idx_t):
    mesh = plsc.VectorSubcoreMesh(
        core_axis_name="c", subcore_axis_name="s",
        num_cores=_NC, num_subcores=_NS)
    return pl.kernel(
        _gather_kernel,
        out_type=jax.ShapeDtypeStruct(
            (_NPOS, _D // 8, _BATCH // 128, 8, 128), jnp.float32),
        mesh=mesh,
        scratch_types=(
            [pltpu.VMEM((_W,), jnp.int32) for _ in range(2)]
            + [pltpu.VMEM((_W, _D), jnp.float32) for _ in range(2)]
            + [pltpu.VMEM((_D // 8, _W // 128, 8, 128), jnp.float32)
               for _ in range(2)]
            + [pltpu.SemaphoreType.DMA for _ in range(6)]
        ),
        compiler_params=pltpu.CompilerParams(
            use_tc_tiling_on_sc=False, needs_layout_passes=False),
    )(embed, idx_t)


def kernel(embed, indices):
    idx_t = jnp.swapaxes(indices, 0, 1).astype(jnp.int32)
    out5 = _lookup(embed, idx_t)
    out3 = out5.transpose(0, 1, 3, 2, 4).reshape(_NPOS, _D, _BATCH)
    return out3.transpose(2, 0, 1)


# speedup vs baseline: 6.0144x; 1.0742x over previous
import jax
import jax.numpy as jnp
from jax import lax
from jax.experimental import pallas as pl
from jax.experimental.pallas import tpu as pltpu
from jax.experimental.pallas import tpu_sc as plsc

_NC = 2
_NS = 16
_NW = _NC * _NS

_D = 32
_BATCH = 16384
_NPOS = 100
_W = _BATCH // _NW
_L = 16


def _gather_kernel(table_hbm, idx_hbm, out_hbm, *scratch):
    idxs = scratch[0:2]
    rows = scratch[2:4]
    valst = scratch[4:6]
    isem = scratch[6:8]
    gsem = scratch[8:10]
    wsem = scratch[10:12]

    wid = lax.axis_index("s") * _NC + lax.axis_index("c")
    b0 = wid * _W

    def idx_copy(p, s):
        return pltpu.make_async_copy(
            idx_hbm.at[p, pl.ds(b0, _W)], idxs[s], isem[s])

    def gather(s):
        return pltpu.make_async_copy(table_hbm.at[idxs[s]], rows[s], gsem[s])

    tcb = wid * (_W // 128)

    def out_copy(p, s):
        return pltpu.make_async_copy(
            valst[s], out_hbm.at[p, :, pl.ds(tcb, _W // 128), :, :], wsem[s])

    iota = lax.iota(jnp.int32, _L)

    for s in range(2):
        idx_copy(s, s).start()
    idx_copy(0, 0).wait()
    gather(0).start()

    @pl.loop(0, _NPOS, step=2)
    def _pair(p0):
        for s in range(2):
            p = p0 + s

            gather(s).wait()

            @pl.when(p0 + 4 <= _NPOS)
            def _():
                idx_copy(p + 2, s).start()

            @pl.when(p + 1 < _NPOS)
            def _():
                idx_copy(p + 1, 1 - s).wait()
                gather(1 - s).start()

            @pl.when(p0 >= 2)
            def _():
                out_copy(p - 2, s).wait()

            @plsc.parallel_loop(0, _W, _L, unroll=2)
            def _blk(bb):
                ridx = bb + iota
                tcl = bb // 128
                cc0 = bb % 128
                for c in range(_D):
                    v = plsc.load_gather(
                        rows[s], [ridx, jnp.full((_L,), c, jnp.int32)])
                    valst[s][c // 8, tcl, c % 8, pl.ds(cc0, _L)] = v

            out_copy(p, s).start()

    for s in range(2):
        out_copy(_NPOS - 2 + s, s).wait()


_ROWS = 1_000_000
_FULL = 999_936
_NCOLS = _FULL // 128
_TAIL = _ROWS - _FULL


def _convert_kernel(embt_hbm, tail_hbm, out_hbm, *scratch):
    blk = scratch[0:2]
    vals = scratch[2:4]
    lsem = scratch[4:6]
    wsem = scratch[6:8]
    tailv = scratch[8]
    tailt = scratch[9]

    wid = lax.axis_index("s") * _NC + lax.axis_index("c")
    base = wid * 244 + jnp.minimum(wid, 4)
    iota = lax.iota(jnp.int32, _L)

    def load(j, s):
        return pltpu.make_async_copy(
            embt_hbm.at[:, pl.ds((base + j) * 128, 128)], blk[s], lsem[s])

    def store(j, s):
        return pltpu.make_async_copy(
            vals[s], out_hbm.at[pl.ds((base + j) * 128 * _D, 128 * _D)],
            wsem[s])

    def transpose(s):
        @plsc.parallel_loop(0, 128, 1, unroll=4)
        def _cc(cc):
            for h in range(2):
                v = plsc.load_gather(
                    blk[s], [h * _L + iota, jnp.full((_L,), 0, jnp.int32) + cc])
                vals[s][pl.ds(cc * _D + h * _L, _L)] = v

    for s in range(2):
        load(s, s).start()

    @pl.loop(0, 244, step=2)
    def _pairs(j0):
        for s in range(2):
            j = j0 + s
            load(j, s).wait()

            @pl.when(j0 >= 2)
            def _():
                store(j - 2, s).wait()

            transpose(s)

            @pl.when(j0 + 4 <= 244)
            def _():
                load(j + 2, s).start()

            store(j, s).start()

    for s in range(2):
        store(242 + s, s).wait()

    @pl.when(wid < 4)
    def _():
        pltpu.sync_copy(
            embt_hbm.at[:, pl.ds((base + 244) * 128, 128)], blk[0])
        transpose(0)
        pltpu.sync_copy(
            vals[0], out_hbm.at[pl.ds((base + 244) * 128 * _D, 128 * _D)])

    @pl.when(wid == 31)
    def _():
        pltpu.sync_copy(tail_hbm, tailv)

        @plsc.parallel_loop(0, _TAIL, 1, unroll=4)
        def _rr(rr):
            for h in range(2):
                v = plsc.load_gather(
                    tailv, [(h * _L + iota) * _TAIL + rr])
                tailt[pl.ds(rr * _D + h * _L, _L)] = v
        pltpu.sync_copy(tailt, out_hbm.at[pl.ds(_FULL * _D, _TAIL * _D)])


@jax.jit
def _convert(embt, tail_flat):
    mesh = plsc.VectorSubcoreMesh(
        core_axis_name="c", subcore_axis_name="s",
        num_cores=_NC, num_subcores=_NS)
    return pl.kernel(
        _convert_kernel,
        out_type=jax.ShapeDtypeStruct((_ROWS * _D,), jnp.float32),
        mesh=mesh,
        scratch_types=(
            [pltpu.VMEM((_D, 128), jnp.float32) for _ in range(2)]
            + [pltpu.VMEM((128 * _D,), jnp.float32) for _ in range(2)]
            + [pltpu.SemaphoreType.DMA for _ in range(4)]
            + [pltpu.VMEM((_TAIL * _D,), jnp.float32),
               pltpu.VMEM((_TAIL * _D,), jnp.float32)]
        ),
        compiler_params=pltpu.CompilerParams(
            use_tc_tiling_on_sc=True, needs_layout_passes=False),
    )(embt, tail_flat)


@jax.jit
def _lookup(embed, idx_t):
    mesh = plsc.VectorSubcoreMesh(
        core_axis_name="c", subcore_axis_name="s",
        num_cores=_NC, num_subcores=_NS)
    return pl.kernel(
        _gather_kernel,
        out_type=jax.ShapeDtypeStruct(
            (_NPOS, _D // 8, _BATCH // 128, 8, 128), jnp.float32),
        mesh=mesh,
        scratch_types=(
            [pltpu.VMEM((_W,), jnp.int32) for _ in range(2)]
            + [pltpu.VMEM((_W, _D), jnp.float32) for _ in range(2)]
            + [pltpu.VMEM((_D // 8, _W // 128, 8, 128), jnp.float32)
               for _ in range(2)]
            + [pltpu.SemaphoreType.DMA for _ in range(6)]
        ),
        compiler_params=pltpu.CompilerParams(
            use_tc_tiling_on_sc=False, needs_layout_passes=False),
    )(embed, idx_t)


def kernel(embed, indices):
    idx_t = jnp.swapaxes(indices, 0, 1).astype(jnp.int32)
    tail_flat = embed[_FULL:].T.reshape(-1)
    table_rm = _convert(embed.T, tail_flat).reshape(_ROWS, _D)
    out5 = _lookup(table_rm, idx_t)
    out3 = out5.transpose(0, 1, 3, 2, 4).reshape(_NPOS, _D, _BATCH)
    return out3.transpose(2, 0, 1)
